# baseline (speedup 1.0000x reference)
import functools

import jax
import jax.numpy as jnp
from jax.experimental import pallas as pl
from jax.experimental.pallas import tpu as pltpu

_NUM_EXPERTS = 8
_TOP_K = 2
_HIDDEN = 768


def _tc_body(h_ref, wt_ref, rw_ref, se_ref, aux_ref, acc_ref, *, n_tokens):
    i = pl.program_id(0)
    n = pl.num_programs(0)

    h = h_ref[...]
    wt = wt_ref[...]
    logits = jnp.dot(h, wt, preferred_element_type=jnp.float32)

    m = jnp.max(logits, axis=1, keepdims=True)
    ex = jnp.exp(logits - m)
    s = jnp.sum(ex, axis=1, keepdims=True)
    p = ex / s

    @pl.when(i == 0)
    def _():
        acc_ref[...] = jnp.zeros_like(acc_ref)

    acc_ref[...] += jnp.sum(p, axis=0, keepdims=True)

    @pl.when(i == n - 1)
    def _():
        usage = acc_ref[...] / n_tokens
        aux_ref[0, 0] = _NUM_EXPERTS * jnp.sum(usage * usage)

    e_iota = jax.lax.broadcasted_iota(jnp.int32, p.shape, 1)
    i1 = jnp.argmax(p, axis=1).astype(jnp.int32)
    v1 = jnp.max(p, axis=1)
    p2 = jnp.where(e_iota == i1[:, None], -1.0, p)
    i2 = jnp.argmax(p2, axis=1).astype(jnp.int32)
    v2 = jnp.max(p2, axis=1)

    denom = v1 + v2
    rw_ref[...] = jnp.stack([v1 / denom, v2 / denom], axis=1)
    se_ref[...] = jnp.stack([i1, i2], axis=1)


@functools.partial(jax.jit, static_argnames=("interpret",))
def kernel(hidden_states, gate_weight, interpret=False):
    b, t, hd = hidden_states.shape
    n_tokens = b * t
    h = hidden_states.reshape(n_tokens, hd)
    wt = gate_weight.T

    blk = 2048
    grid = (n_tokens // blk,)

    rw, se, aux = pl.pallas_call(
        functools.partial(_tc_body, n_tokens=n_tokens),
        grid=grid,
        in_specs=[
            pl.BlockSpec((blk, hd), lambda i: (i, 0)),
            pl.BlockSpec((hd, _NUM_EXPERTS), lambda i: (0, 0)),
        ],
        out_specs=[
            pl.BlockSpec((blk, _TOP_K), lambda i: (i, 0)),
            pl.BlockSpec((blk, _TOP_K), lambda i: (i, 0)),
            pl.BlockSpec(memory_space=pltpu.SMEM),
        ],
        out_shape=[
            jax.ShapeDtypeStruct((n_tokens, _TOP_K), jnp.float32),
            jax.ShapeDtypeStruct((n_tokens, _TOP_K), jnp.int32),
            jax.ShapeDtypeStruct((1, 1), jnp.float32),
        ],
        scratch_shapes=[pltpu.VMEM((1, _NUM_EXPERTS), jnp.float32)],
        interpret=interpret,
    )(h, wt)

    return rw, se, aux.reshape(())
